# baseline (speedup 1.0000x reference)
import functools

import jax
import jax.numpy as jnp
from jax import lax
from jax.experimental import pallas as pl
from jax.experimental.pallas import tpu as pltpu
from jax.experimental.pallas import tpu_sc as plsc

BATCH = 16384
DIM = 64
NC = 2
NS = 16
L = 16
NW = NC * NS
BPW = BATCH // NW
CH = 128
NCH = BPW // CH

_mesh = plsc.VectorSubcoreMesh(core_axis_name="c", subcore_axis_name="s")


@functools.partial(
    pl.kernel,
    mesh=_mesh,
    out_type=jax.ShapeDtypeStruct((BATCH, DIM), jnp.float32),
    scratch_types=[
        pltpu.VMEM_SHARED((NW, 2, BPW), jnp.int32),
        pltpu.SMEM((BPW,), jnp.int32),
        pltpu.VMEM((BPW,), jnp.int32),
        pltpu.VMEM((BPW,), jnp.int32),
        pltpu.VMEM((CH, 2 * DIM), jnp.float32),
        pltpu.VMEM((CH, 2 * DIM), jnp.float32),
        pltpu.VMEM((BPW, 2 * DIM), jnp.float32),
        pltpu.VMEM((CH, DIM), jnp.float32),
        pltpu.SemaphoreType.DMA,
        pltpu.SemaphoreType.DMA,
        pltpu.SemaphoreType.DMA,
    ],
)
def _distmult_sc(e1_hbm, r_hbm, ent_hbm, rel_hbm, out_hbm,
                 idx_sh, e_idx, eg_v, r_idx_v, e_rows0, e_rows1, r_rows,
                 out_v, sem_e0, sem_e1, sem_r):
    wid = lax.axis_index("s") * NC + lax.axis_index("c")
    base = wid * BPW
    pltpu.sync_copy(e1_hbm.at[pl.ds(base, BPW)], idx_sh.at[wid, 0])
    pltpu.sync_copy(idx_sh.at[wid, 0], e_idx)
    pltpu.sync_copy(e1_hbm.at[pl.ds(base, BPW)], eg_v)
    pltpu.sync_copy(r_hbm.at[pl.ds(base, BPW)], r_idx_v)

    def shift(i, _):
        sl = pl.ds(i * L, L)
        eg_v[sl] = lax.shift_right_logical(eg_v[sl], 1)
        return ()

    lax.fori_loop(0, BPW // L, shift, ())

    cr = pltpu.async_copy(rel_hbm.at[r_idx_v], r_rows, sem_r)

    e_bufs = (e_rows0, e_rows1)
    e_sems = (sem_e0, sem_e1)

    def fire(c):
        return pltpu.async_copy(
            ent_hbm.at[eg_v.at[pl.ds(c * CH, CH)]], e_bufs[c % 2],
            e_sems[c % 2])

    def mult_store(c, buf):
        def body(i, _):
            half = lax.rem(e_idx[c * CH + i], 2) * DIM
            for j in range(DIM // L):
                out_v[i, pl.ds(j * L, L)] = (
                    buf[i, pl.ds(half + j * L, L)]
                    * r_rows[c * CH + i, pl.ds(j * L, L)])
            return ()
        lax.fori_loop(0, CH, body, ())
        pltpu.sync_copy(out_v, out_hbm.at[pl.ds(base + c * CH, CH)])

    handles = [fire(0)]
    cr.wait()
    for c in range(NCH):
        if c + 1 < NCH:
            handles.append(fire(c + 1))
        handles[c].wait()
        mult_store(c, e_bufs[c % 2])


def kernel(e1, r, entity_table, relation_table):
    rel128 = jnp.pad(relation_table, ((0, 0), (0, DIM)))
    ent2 = entity_table.reshape(entity_table.shape[0] // 2, 2 * DIM)
    return _distmult_sc(e1.astype(jnp.int32), r.astype(jnp.int32),
                        ent2, rel128)


# speedup vs baseline: 2.5323x; 2.5323x over previous
import functools

import jax
import jax.numpy as jnp
from jax import lax
from jax.experimental import pallas as pl
from jax.experimental.pallas import tpu as pltpu
from jax.experimental.pallas import tpu_sc as plsc

BATCH = 16384
DIM = 64
NC = 2
NS = 16
L = 16
NW = NC * NS
BPW = BATCH // NW
CH = 128
NCH = BPW // CH

_mesh = plsc.VectorSubcoreMesh(core_axis_name="c", subcore_axis_name="s")


@functools.partial(
    pl.kernel,
    mesh=_mesh,
    compiler_params=pltpu.CompilerParams(skip_device_barrier=True),
    out_type=jax.ShapeDtypeStruct((BATCH, DIM), jnp.float32),
    scratch_types=[
        pltpu.VMEM_SHARED((NW, 2, BPW), jnp.int32),
        pltpu.SMEM((BPW,), jnp.int32),
        pltpu.VMEM((BPW,), jnp.int32),
        pltpu.VMEM((CH, DIM), jnp.float32),
        pltpu.VMEM((CH, DIM), jnp.float32),
        pltpu.VMEM((BPW, 2 * DIM), jnp.float32),
        pltpu.SemaphoreType.DMA,
        pltpu.SemaphoreType.DMA,
        pltpu.SemaphoreType.DMA,
        pltpu.SemaphoreType.DMA,
        pltpu.SemaphoreType.DMA,
        pltpu.SemaphoreType.DMA,
        pltpu.SemaphoreType.DMA,
        pltpu.SemaphoreType.DMA,
        pltpu.SemaphoreType.DMA,
    ],
)
def _distmult_sc(e1_hbm, r_hbm, ent_hbm, rel_hbm, out_hbm,
                 idx_sh, e_idx, r_idx_v, e_rows0, e_rows1, r_rows,
                 se0, se1, se2, se3, se4, se5, se6, se7, sem_r):
    wid = lax.axis_index("s") * NC + lax.axis_index("c")
    base = wid * BPW
    pltpu.sync_copy(e1_hbm.at[pl.ds(base, BPW)], idx_sh.at[wid, 0])
    pltpu.sync_copy(idx_sh.at[wid, 0], e_idx)
    pltpu.sync_copy(r_hbm.at[pl.ds(base, BPW)], r_idx_v)
    cr = pltpu.async_copy(rel_hbm.at[r_idx_v], r_rows, sem_r)

    e_bufs = (e_rows0, e_rows1)
    e_sems = ((se0, se1, se2, se3), (se4, se5, se6, se7))
    NSEM = 4

    def fire(c, buf, sems):
        def body(i, _):
            for u in range(NSEM):
                k = i * NSEM + u
                idx = e_idx[c * CH + k]
                g = lax.shift_right_logical(idx, 4)
                s = lax.rem(idx, 16)
                pltpu.async_copy(ent_hbm.at[g, s], buf.at[k], sems[u])
            return ()
        lax.fori_loop(0, CH // NSEM, body, ())

    def drain_mult_store(c, buf, sems):
        for u in range(NSEM):
            pltpu.make_async_copy(ent_hbm.at[0, pl.ds(0, CH // NSEM)],
                                  buf.at[pl.ds(0, CH // NSEM)], sems[u]).wait()

        def body(i, _):
            for j in range(DIM // L):
                sl = pl.ds(j * L, L)
                buf[i, sl] = buf[i, sl] * r_rows[c * CH + i, sl]
            return ()
        lax.fori_loop(0, CH, body, ())
        pltpu.sync_copy(buf, out_hbm.at[pl.ds(base + c * CH, CH)])

    fire(0, e_bufs[0], e_sems[0])
    cr.wait()
    for c in range(NCH):
        if c + 1 < NCH:
            fire(c + 1, e_bufs[(c + 1) % 2], e_sems[(c + 1) % 2])
        drain_mult_store(c, e_bufs[c % 2], e_sems[c % 2])


def kernel(e1, r, entity_table, relation_table):
    rel128 = jnp.pad(relation_table, ((0, 0), (0, DIM)))
    ent3 = entity_table.reshape(entity_table.shape[0] // 16, 16, DIM)
    return _distmult_sc(e1.astype(jnp.int32), r.astype(jnp.int32),
                        ent3, rel128)
